# baseline (speedup 1.0000x reference)
import numpy as np
import jax
import jax.numpy as jnp
from jax import lax
from jax.experimental import pallas as pl
from jax.experimental.pallas import tpu as pltpu

_K = 32
_LANES = 2048
_TINY = float(np.finfo(np.float32).tiny)
_M32 = 0xFFFFFFFF


def _tf_block(k0, k1, x0, x1):
    ks0, ks1 = k0, k1
    ks2 = ks0 ^ ks1 ^ 0x1BD11BDA
    rot0 = (13, 15, 26, 6)
    rot1 = (17, 29, 16, 24)

    def rnds(x0, x1, rots):
        for r in rots:
            x0 = (x0 + x1) & _M32
            x1 = ((x1 << r) | (x1 >> (32 - r))) & _M32
            x1 = x1 ^ x0
        return x0, x1

    x0 = (x0 + ks0) & _M32
    x1 = (x1 + ks1) & _M32
    x0, x1 = rnds(x0, x1, rot0)
    x0 = (x0 + ks1) & _M32
    x1 = (x1 + ks2 + 1) & _M32
    x0, x1 = rnds(x0, x1, rot1)
    x0 = (x0 + ks2) & _M32
    x1 = (x1 + ks0 + 2) & _M32
    x0, x1 = rnds(x0, x1, rot0)
    x0 = (x0 + ks0) & _M32
    x1 = (x1 + ks1 + 3) & _M32
    x0, x1 = rnds(x0, x1, rot1)
    x0 = (x0 + ks1) & _M32
    x1 = (x1 + ks2 + 4) & _M32
    x0, x1 = rnds(x0, x1, rot0)
    x0 = (x0 + ks2) & _M32
    x1 = (x1 + ks0 + 5) & _M32
    return x0, x1


_KX = _tf_block(0, 1, 0, 0)
_KY = _tf_block(0, 1, 0, 1)


def _tf_block_vec(k0, k1, x1):
    u = lambda c: jnp.uint32(c & _M32)
    ks0, ks1 = k0, k1
    ks2 = k0 ^ k1 ^ 0x1BD11BDA

    def rnds(x0, x1, rots):
        for r in rots:
            x0 = x0 + x1
            x1 = (x1 << u(r)) | (x1 >> u(32 - r))
            x1 = x1 ^ x0
        return x0, x1

    x0 = x1 + u(ks0)
    x1 = ((x1 << u(13)) | (x1 >> u(19))) ^ x0
    x0, x1 = rnds(x0, x1, (15, 26, 6))
    x0 = x0 + u(ks1)
    x1 = x1 + u(ks2 + 1)
    x0, x1 = rnds(x0, x1, (17, 29, 16, 24))
    x0 = x0 + u(ks2)
    x1 = x1 + u(ks0 + 2)
    x0, x1 = rnds(x0, x1, (13, 15, 26, 6))
    x0 = x0 + u(ks0)
    x1 = x1 + u(ks1 + 3)
    x0, x1 = rnds(x0, x1, (17, 29, 16, 24))
    x0 = x0 + u(ks1)
    x1 = x1 + u(ks2 + 4)
    x0, x1 = rnds(x0, x1, (13, 15, 26, 6))
    x0 = x0 + u(ks2)
    x1 = x1 + u(ks0 + 5)
    return x0, x1


def _sample_body(x_ref, o_ref, sval, sidx, *, key, B, V):
    i = pl.program_id(0)
    j = pl.program_id(1)
    nvt = pl.num_programs(1)

    v32 = j * _LANES + lax.broadcasted_iota(jnp.int32, (8, _LANES), 1)
    nrv = jnp.where(v32 < V, -1.0 / x_ref[...], -jnp.inf)

    @pl.when(j == 0)
    def _():
        sval[...] = jnp.full((_K, 8, _LANES), jnp.inf, jnp.float32)
        sidx[...] = jnp.zeros((_K, 8, _LANES), jnp.int32)

    b32 = i * 8 + lax.broadcasted_iota(jnp.int32, (8, _LANES), 0)
    base = (b32 * V + v32).astype(jnp.uint32)

    def draw_step(kk, _):
        c = kk.astype(jnp.uint32) * jnp.uint32(B * V) + jnp.uint32(key[1])
        x1 = base + c
        w0, w1 = _tf_block_vec(key[0], key[1], x1)
        bits = w0 ^ w1
        fb = lax.bitcast_convert_type(
            (bits >> jnp.uint32(9)) | jnp.uint32(0x3F800000), jnp.float32)
        u = fb - 1.0
        t = jnp.log(u) * nrv
        cur = sval[kk]
        upd = t < cur
        sval[kk] = jnp.where(upd, t, cur)
        sidx[kk] = jnp.where(upd, v32, sidx[kk])
        return 0

    lax.fori_loop(0, _K, draw_step, 0)

    @pl.when(j == nvt - 1)
    def _():
        lane = lax.broadcasted_iota(jnp.int32, (8, _K), 1)

        def fin_step(kk, acc):
            tv = sval[kk]
            m = jnp.min(tv, axis=1, keepdims=True)
            idx = jnp.min(jnp.where(tv == m, sidx[kk], jnp.int32(2**31 - 1)),
                          axis=1, keepdims=True)
            return jnp.where(lane == kk, idx, acc)

        o_ref[...] = lax.fori_loop(0, _K, fin_step,
                                   jnp.zeros((8, _K), jnp.int32))


def _sample(x, key):
    B, V = x.shape
    nvt = pl.cdiv(V, _LANES)
    import functools
    body = functools.partial(_sample_body, key=key, B=B, V=V)
    return pl.pallas_call(
        body,
        grid=(B // 8, nvt),
        in_specs=[pl.BlockSpec((8, _LANES), lambda i, j: (i, j))],
        out_specs=pl.BlockSpec((8, _K), lambda i, j: (i, 0)),
        out_shape=jax.ShapeDtypeStruct((B, _K), jnp.int32),
        scratch_shapes=[
            pltpu.VMEM((_K, 8, _LANES), jnp.float32),
            pltpu.VMEM((_K, 8, _LANES), jnp.int32),
        ],
        compiler_params=pltpu.CompilerParams(
            dimension_semantics=("parallel", "arbitrary")),
    )(x)


def _matmul_body(sx_ref, syt_ref, a_ref):
    sx = sx_ref[...]
    syt = syt_ref[...]
    acc = sx[:, 0:1] * syt[0:1, :]
    for k in range(1, _K):
        acc = acc + sx[:, k:k + 1] * syt[k:k + 1, :]
    a_ref[...] = acc


def _matmul(sx, syt):
    Bx = sx.shape[0]
    By = syt.shape[1]
    return pl.pallas_call(
        _matmul_body,
        out_shape=jax.ShapeDtypeStruct((Bx, By), jnp.int32),
    )(sx, syt)


def kernel(x, y):
    sx = _sample(x, _KX)
    sy = _sample(y, _KY)
    return _matmul(sx, sy.T)


# speedup vs baseline: 1.1495x; 1.1495x over previous
import numpy as np
import jax
import jax.numpy as jnp
from jax import lax
from jax.experimental import pallas as pl
from jax.experimental.pallas import tpu as pltpu

_K = 32
_LANES = 2048
_TINY = float(np.finfo(np.float32).tiny)
_M32 = 0xFFFFFFFF


def _tf_block(k0, k1, x0, x1):
    ks0, ks1 = k0, k1
    ks2 = ks0 ^ ks1 ^ 0x1BD11BDA
    rot0 = (13, 15, 26, 6)
    rot1 = (17, 29, 16, 24)

    def rnds(x0, x1, rots):
        for r in rots:
            x0 = (x0 + x1) & _M32
            x1 = ((x1 << r) | (x1 >> (32 - r))) & _M32
            x1 = x1 ^ x0
        return x0, x1

    x0 = (x0 + ks0) & _M32
    x1 = (x1 + ks1) & _M32
    x0, x1 = rnds(x0, x1, rot0)
    x0 = (x0 + ks1) & _M32
    x1 = (x1 + ks2 + 1) & _M32
    x0, x1 = rnds(x0, x1, rot1)
    x0 = (x0 + ks2) & _M32
    x1 = (x1 + ks0 + 2) & _M32
    x0, x1 = rnds(x0, x1, rot0)
    x0 = (x0 + ks0) & _M32
    x1 = (x1 + ks1 + 3) & _M32
    x0, x1 = rnds(x0, x1, rot1)
    x0 = (x0 + ks1) & _M32
    x1 = (x1 + ks2 + 4) & _M32
    x0, x1 = rnds(x0, x1, rot0)
    x0 = (x0 + ks2) & _M32
    x1 = (x1 + ks0 + 5) & _M32
    return x0, x1


_KX = _tf_block(0, 1, 0, 0)
_KY = _tf_block(0, 1, 0, 1)


def _tf_block_vec(k0, k1, x1):
    u = lambda c: jnp.uint32(c & _M32)
    ks0, ks1 = k0, k1
    ks2 = k0 ^ k1 ^ 0x1BD11BDA

    def rnds(x0, x1, rots):
        for r in rots:
            x0 = x0 + x1
            x1 = (x1 << u(r)) | (x1 >> u(32 - r))
            x1 = x1 ^ x0
        return x0, x1

    x0 = x1 + u(ks0)
    x1 = ((x1 << u(13)) | (x1 >> u(19))) ^ x0
    x0, x1 = rnds(x0, x1, (15, 26, 6))
    x0 = x0 + u(ks1)
    x1 = x1 + u(ks2 + 1)
    x0, x1 = rnds(x0, x1, (17, 29, 16, 24))
    x0 = x0 + u(ks2)
    x1 = x1 + u(ks0 + 2)
    x0, x1 = rnds(x0, x1, (13, 15, 26, 6))
    x0 = x0 + u(ks0)
    x1 = x1 + u(ks1 + 3)
    x0, x1 = rnds(x0, x1, (17, 29, 16, 24))
    x0 = x0 + u(ks1)
    x1 = x1 + u(ks2 + 4)
    x0, x1 = rnds(x0, x1, (13, 15, 26, 6))
    x0 = x0 + u(ks2)
    x1 = x1 + u(ks0 + 5)
    return x0, x1


def _sample_body(x_ref, o_ref, sval, sidx, *, key, B, V):
    i = pl.program_id(0)
    j = pl.program_id(1)
    nvt = pl.num_programs(1)

    v32 = j * _LANES + lax.broadcasted_iota(jnp.int32, (8, _LANES), 1)
    nrv = jnp.where(v32 < V, -1.0 / x_ref[...], -jnp.inf)

    @pl.when(j == 0)
    def _():
        sval[...] = jnp.full((_K, 8, _LANES), jnp.inf, jnp.float32)
        sidx[...] = jnp.zeros((_K, 8, _LANES), jnp.int32)

    b32 = i * 8 + lax.broadcasted_iota(jnp.int32, (8, _LANES), 0)
    base = (b32 * V + v32).astype(jnp.uint32)

    for kk in range(_K):
        c = jnp.uint32((kk * B * V + key[1]) & _M32)
        x1 = base + c
        w0, w1 = _tf_block_vec(key[0], key[1], x1)
        bits = w0 ^ w1
        fb = lax.bitcast_convert_type(
            (bits >> jnp.uint32(9)) | jnp.uint32(0x3F800000), jnp.float32)
        u = fb - 1.0
        t = jnp.log(u) * nrv
        cur = sval[kk]
        upd = t < cur
        sval[kk] = jnp.where(upd, t, cur)
        sidx[kk] = jnp.where(upd, v32, sidx[kk])

    @pl.when(j == nvt - 1)
    def _():
        lane = lax.broadcasted_iota(jnp.int32, (8, _K), 1)

        def fin_step(kk, acc):
            tv = sval[kk]
            m = jnp.min(tv, axis=1, keepdims=True)
            idx = jnp.min(jnp.where(tv == m, sidx[kk], jnp.int32(2**31 - 1)),
                          axis=1, keepdims=True)
            return jnp.where(lane == kk, idx, acc)

        o_ref[...] = lax.fori_loop(0, _K, fin_step,
                                   jnp.zeros((8, _K), jnp.int32))


def _sample(x, key):
    B, V = x.shape
    nvt = pl.cdiv(V, _LANES)
    import functools
    body = functools.partial(_sample_body, key=key, B=B, V=V)
    return pl.pallas_call(
        body,
        grid=(B // 8, nvt),
        in_specs=[pl.BlockSpec((8, _LANES), lambda i, j: (i, j))],
        out_specs=pl.BlockSpec((8, _K), lambda i, j: (i, 0)),
        out_shape=jax.ShapeDtypeStruct((B, _K), jnp.int32),
        scratch_shapes=[
            pltpu.VMEM((_K, 8, _LANES), jnp.float32),
            pltpu.VMEM((_K, 8, _LANES), jnp.int32),
        ],
        compiler_params=pltpu.CompilerParams(
            dimension_semantics=("parallel", "arbitrary")),
    )(x)


def _matmul_body(sx_ref, syt_ref, a_ref):
    sx = sx_ref[...]
    syt = syt_ref[...]
    acc = sx[:, 0:1] * syt[0:1, :]
    for k in range(1, _K):
        acc = acc + sx[:, k:k + 1] * syt[k:k + 1, :]
    a_ref[...] = acc


def _matmul(sx, syt):
    Bx = sx.shape[0]
    By = syt.shape[1]
    return pl.pallas_call(
        _matmul_body,
        out_shape=jax.ShapeDtypeStruct((Bx, By), jnp.int32),
    )(sx, syt)


def kernel(x, y):
    sx = _sample(x, _KX)
    sy = _sample(y, _KY)
    return _matmul(sx, sy.T)
